# baseline (speedup 1.0000x reference)
import functools

import jax
import jax.numpy as jnp
from jax import lax
from jax.experimental import pallas as pl
from jax.experimental.pallas import tpu as pltpu
from jax.experimental.pallas import tpu_sc as plsc

_B = 4
_N = 2048
_K = 10
_NC = 2
_NS = 16
_NW = _NC * _NS
_CPB = _NW // _B
_RPW = _N // _CPB
_IL = 8
_NCHUNK = _N // 16


def _sqrt16(x):
    i = lax.bitcast_convert_type(x, jnp.int32)
    y = lax.bitcast_convert_type(jnp.int32(0x5F3759DF) - (i >> 1), jnp.float32)
    for _ in range(3):
        y = y * (jnp.float32(1.5) - jnp.float32(0.5) * x * y * y)
    return x * y


_mesh = plsc.VectorSubcoreMesh(
    core_axis_name="c", subcore_axis_name="s", num_cores=_NC, num_subcores=_NS
)


@functools.partial(
    pl.kernel,
    out_type=jax.ShapeDtypeStruct((_NW, 16), jnp.float32),
    mesh=_mesh,
    compiler_params=pltpu.CompilerParams(needs_layout_passes=False),
    scratch_types=[
        pltpu.VMEM((_N,), jnp.float32),
        pltpu.VMEM((_N,), jnp.float32),
        pltpu.VMEM((_N,), jnp.float32),
        pltpu.VMEM((_N,), jnp.float32),
        pltpu.VMEM((16,), jnp.float32),
    ],
)
def _sc_topk(x_hbm, y_hbm, z_hbm, r_hbm, out_hbm, cx, cy, cz, rr, ostage):
    wid = lax.axis_index("s") * _NC + lax.axis_index("c")
    b = wid // _CPB
    base_row = (wid % _CPB) * _RPW
    pltpu.sync_copy(x_hbm.at[b], cx)
    pltpu.sync_copy(y_hbm.at[b], cy)
    pltpu.sync_copy(z_hbm.at[b], cz)
    pltpu.sync_copy(r_hbm.at[b], rr)

    lane = lax.iota(jnp.int32, 16)
    inf = jnp.float32(jnp.inf)

    z16 = jnp.zeros((16,), jnp.float32)

    def row_group16(t, carry):
        s_acc, s2_acc = carry
        g16 = base_row + t * 16
        rx16 = cx[pl.ds(g16, 16)]
        ry16 = cy[pl.ds(g16, 16)]
        rz16 = cz[pl.ds(g16, 16)]
        rr16 = rr[pl.ds(g16, 16)]
        topv = z16
        for sub in range(16 // _IL):
            rows = [g16 + sub * _IL + j for j in range(_IL)]
            sx = [jnp.full((16,), rx16[sub * _IL + j]) for j in range(_IL)]
            sy = [jnp.full((16,), ry16[sub * _IL + j]) for j in range(_IL)]
            sz = [jnp.full((16,), rz16[sub * _IL + j]) for j in range(_IL)]

            def chunk_step(c, ks):
                off = c * 16
                xv = cx[pl.ds(off, 16)]
                yv = cy[pl.ds(off, 16)]
                zv = cz[pl.ds(off, 16)]
                rv = rr[pl.ds(off, 16)]
                out = []
                for j in range(_IL):
                    kk, kv = ks[2 * j], ks[2 * j + 1]
                    dx = xv - sx[j]
                    dy = yv - sy[j]
                    dz = zv - sz[j]
                    d2 = dx * dx + dy * dy + dz * dz
                    sk, sv = plsc.sort_key_val(d2, rv)
                    sel = kk <= sk
                    lok = jnp.where(sel, kk, sk)
                    lov = jnp.where(sel, kv, sv)
                    kk, kv = plsc.sort_key_val(lok, lov, descending=True)
                    out += [kk, kv]
                return tuple(out)

            k0 = (jnp.full((16,), inf), z16) * _IL
            ks = lax.fori_loop(0, _NCHUNK, chunk_step, k0)
            for j in range(_IL):
                g = _sqrt16(ks[2 * j]) - ks[2 * j + 1]
                g = jnp.where((lane >= 16 - 1 - _K) & (lane < 15), g, inf)
                top = jnp.min(g) - rr16[sub * _IL + j]
                topv = jnp.where(lane == sub * _IL + j, jnp.full((16,), top), topv)
        return s_acc + topv, s2_acc + topv * topv

    sv, s2v = lax.fori_loop(0, _RPW // 16, row_group16, (z16, z16))
    s = jnp.sum(sv)
    s2 = jnp.sum(s2v)
    ostage[...] = jnp.where(lane == 0, s, jnp.where(lane == 1, s2, jnp.float32(0.0)))
    pltpu.sync_copy(ostage, out_hbm.at[wid])


def _fin_body(p_ref, out_ref):
    p = p_ref[...]
    ri = lax.broadcasted_iota(jnp.int32, (_NW, 16), 0)
    ci = lax.broadcasted_iota(jnp.int32, (_NW, 16), 1)
    n = jnp.float32(_N)
    tot = jnp.float32(0.0)
    for b in range(_B):
        in_b = ri // _CPB == b
        s = jnp.sum(jnp.where(in_b & (ci == 0), p, 0.0))
        s2 = jnp.sum(jnp.where(in_b & (ci == 1), p, 0.0))
        var = (s2 - s * s / n) / (n - 1.0)
        tot = tot + var
    out_ref[...] = jnp.full((8, 128), tot / jnp.float32(_B))


def _finalize(partials):
    out = pl.pallas_call(
        _fin_body,
        out_shape=jax.ShapeDtypeStruct((8, 128), jnp.float32),
    )(partials)
    return out[0, 0]


@jax.jit
def kernel(spheres):
    coords = jnp.transpose(spheres, (0, 2, 1))
    cx = coords[:, 0]
    cy = coords[:, 1]
    cz = coords[:, 2]
    rr = coords[:, 3]
    partials = _sc_topk(cx, cy, cz, rr)
    return _finalize(partials)


# speedup vs baseline: 1.2089x; 1.2089x over previous
import functools

import jax
import jax.numpy as jnp
from jax import lax
from jax.experimental import pallas as pl
from jax.experimental.pallas import tpu as pltpu
from jax.experimental.pallas import tpu_sc as plsc

_B = 4
_N = 2048
_K = 10
_NC = 2
_NS = 16
_NW = _NC * _NS
_CPB = _NW // _B
_RPW = _N // _CPB
_IL = 4
_NCHUNK = _N // 16


def _sqrt16(x):
    i = lax.bitcast_convert_type(x, jnp.int32)
    y = lax.bitcast_convert_type(jnp.int32(0x5F3759DF) - (i >> 1), jnp.float32)
    for _ in range(3):
        y = y * (jnp.float32(1.5) - jnp.float32(0.5) * x * y * y)
    return x * y


_mesh = plsc.VectorSubcoreMesh(
    core_axis_name="c", subcore_axis_name="s", num_cores=_NC, num_subcores=_NS
)


@functools.partial(
    pl.kernel,
    out_type=jax.ShapeDtypeStruct((_NW, 16), jnp.float32),
    mesh=_mesh,
    compiler_params=pltpu.CompilerParams(needs_layout_passes=False),
    scratch_types=[
        pltpu.VMEM((_N,), jnp.float32),
        pltpu.VMEM((_N,), jnp.float32),
        pltpu.VMEM((_N,), jnp.float32),
        pltpu.VMEM((_N,), jnp.float32),
        pltpu.VMEM((_N,), jnp.float32),
        pltpu.VMEM((16,), jnp.float32),
    ],
)
def _sc_topk(x_hbm, y_hbm, z_hbm, r_hbm, out_hbm, cx, cy, cz, rr, c2, ostage):
    wid = lax.axis_index("s") * _NC + lax.axis_index("c")
    b = wid // _CPB
    base_row = (wid % _CPB) * _RPW
    pltpu.sync_copy(x_hbm.at[b], cx)
    pltpu.sync_copy(y_hbm.at[b], cy)
    pltpu.sync_copy(z_hbm.at[b], cz)
    pltpu.sync_copy(r_hbm.at[b], rr)

    lane = lax.iota(jnp.int32, 16)
    inf = jnp.float32(jnp.inf)

    z16 = jnp.zeros((16,), jnp.float32)

    def c2_step(c, carry):
        off = c * 16
        xv = cx[pl.ds(off, 16)]
        yv = cy[pl.ds(off, 16)]
        zv = cz[pl.ds(off, 16)]
        c2[pl.ds(off, 16)] = xv * xv + yv * yv + zv * zv
        return carry

    lax.fori_loop(0, _NCHUNK, c2_step, jnp.int32(0))

    def row_group16(t, carry):
        s_acc, s2_acc = carry
        g16 = base_row + t * 16
        rx16 = cx[pl.ds(g16, 16)]
        ry16 = cy[pl.ds(g16, 16)]
        rz16 = cz[pl.ds(g16, 16)]
        rr16 = rr[pl.ds(g16, 16)]
        topv = z16
        for sub in range(16 // _IL):
            idxs = [sub * _IL + j for j in range(_IL)]
            sx = [jnp.full((16,), rx16[i]) for i in idxs]
            sy = [jnp.full((16,), ry16[i]) for i in idxs]
            sz = [jnp.full((16,), rz16[i]) for i in idxs]
            nx = [jnp.float32(-2.0) * v for v in sx]
            ny = [jnp.float32(-2.0) * v for v in sy]
            nz = [jnp.float32(-2.0) * v for v in sz]
            rn2 = [(a * a + b * b) + c * c for a, b, c in zip(sx, sy, sz)]

            def chunk_step(c, ks):
                off = c * 16
                xv = cx[pl.ds(off, 16)]
                yv = cy[pl.ds(off, 16)]
                zv = cz[pl.ds(off, 16)]
                rv = rr[pl.ds(off, 16)]
                c2v = c2[pl.ds(off, 16)]
                out = []
                for j in range(_IL):
                    kk, kv = ks[2 * j], ks[2 * j + 1]
                    key = ((c2v + xv * nx[j]) + yv * ny[j]) + zv * nz[j]
                    sk, sv = plsc.sort_key_val(key, rv)
                    sel = kk <= sk
                    lok = jnp.where(sel, kk, sk)
                    lov = jnp.where(sel, kv, sv)
                    kk, kv = plsc.sort_key_val(lok, lov, descending=True)
                    out += [kk, kv]
                return tuple(out)

            k0 = (jnp.full((16,), inf), z16) * _IL
            ks = lax.fori_loop(0, _NCHUNK, chunk_step, k0)
            for j in range(_IL):
                d2c = jnp.maximum(ks[2 * j] + rn2[j], jnp.float32(0.0))
                g = _sqrt16(d2c) - ks[2 * j + 1]
                g = jnp.where((lane >= 16 - 1 - _K) & (lane < 15), g, inf)
                top = jnp.min(g) - rr16[idxs[j]]
                topv = jnp.where(lane == idxs[j], jnp.full((16,), top), topv)
        return s_acc + topv, s2_acc + topv * topv

    sv, s2v = lax.fori_loop(0, _RPW // 16, row_group16, (z16, z16))
    s = jnp.sum(sv)
    s2 = jnp.sum(s2v)
    ostage[...] = jnp.where(lane == 0, s, jnp.where(lane == 1, s2, jnp.float32(0.0)))
    pltpu.sync_copy(ostage, out_hbm.at[wid])


def _fin_body(p_ref, out_ref):
    p = p_ref[...]
    ri = lax.broadcasted_iota(jnp.int32, (_NW, 16), 0)
    ci = lax.broadcasted_iota(jnp.int32, (_NW, 16), 1)
    n = jnp.float32(_N)
    tot = jnp.float32(0.0)
    for b in range(_B):
        in_b = ri // _CPB == b
        s = jnp.sum(jnp.where(in_b & (ci == 0), p, 0.0))
        s2 = jnp.sum(jnp.where(in_b & (ci == 1), p, 0.0))
        var = (s2 - s * s / n) / (n - 1.0)
        tot = tot + var
    out_ref[...] = jnp.full((8, 128), tot / jnp.float32(_B))


def _finalize(partials):
    out = pl.pallas_call(
        _fin_body,
        out_shape=jax.ShapeDtypeStruct((8, 128), jnp.float32),
    )(partials)
    return out[0, 0]


@jax.jit
def kernel(spheres):
    coords = jnp.transpose(spheres, (0, 2, 1))
    cx = coords[:, 0]
    cy = coords[:, 1]
    cz = coords[:, 2]
    rr = coords[:, 3]
    partials = _sc_topk(cx, cy, cz, rr)
    return _finalize(partials)
